# baseline (speedup 1.0000x reference)
import functools

import jax
import jax.numpy as jnp
from jax.experimental import pallas as pl
from jax.experimental.pallas import tpu as pltpu

_LANE_PAD = 128
_HEAD = 2
_OUT = 4
_INV_TEMP = 1.0 / 5.0
_TANH_C = 2.5
_BT = 256


def _ctrl_kernel(idx_ref, h_ref, c_ref, emb_ref, w_ref, b_ref,
                 decw_ref, decb_ref, logits_ref, hx_ref, cx_ref, *, hid):
    n_emb = emb_ref.shape[0]
    wx = w_ref[:hid, :].astype(jnp.bfloat16)
    wh = w_ref[hid:, :].astype(jnp.bfloat16)

    eg = jnp.dot(emb_ref[...].astype(jnp.bfloat16), wx,
                 preferred_element_type=jnp.float32)
    eg = (eg + b_ref[...]).astype(jnp.bfloat16)
    onehot = (idx_ref[...] == jax.lax.broadcasted_iota(
        jnp.int32, (1, n_emb), 1)).astype(jnp.bfloat16)
    gx = jnp.dot(onehot, eg, preferred_element_type=jnp.float32)

    gh = jnp.dot(h_ref[...].astype(jnp.bfloat16), wh,
                 preferred_element_type=jnp.float32)
    gates = gx + gh

    i_g = jax.nn.sigmoid(gates[:, 0 * hid:1 * hid])
    f_g = jax.nn.sigmoid(gates[:, 1 * hid:2 * hid])
    g_g = jnp.tanh(gates[:, 2 * hid:3 * hid])
    o_g = jax.nn.sigmoid(gates[:, 3 * hid:4 * hid])

    cx = f_g * c_ref[...] + i_g * g_g
    hx = o_g * jnp.tanh(cx)

    logits = (jnp.dot(hx.astype(jnp.bfloat16),
                      decw_ref[...].astype(jnp.bfloat16),
                      preferred_element_type=jnp.float32)
              + decb_ref[...])
    logits = _TANH_C * jnp.tanh(logits * _INV_TEMP)
    logits_ref[...] = logits[:, :_OUT]
    hx_ref[...] = hx
    cx_ref[...] = cx


@functools.partial(jax.jit, static_argnames=("batch", "hid", "bt"))
def _run(idx2, h0, c0, embedding, w_lstm, b_lstm, dec_w_pad, dec_b_pad,
         batch, hid, bt):
    kernel_body = functools.partial(_ctrl_kernel, hid=hid)
    n_emb = embedding.shape[0]
    return pl.pallas_call(
        kernel_body,
        out_shape=(
            jax.ShapeDtypeStruct((batch, _OUT), jnp.float32),
            jax.ShapeDtypeStruct((batch, hid), jnp.float32),
            jax.ShapeDtypeStruct((batch, hid), jnp.float32),
        ),
        grid=(batch // bt,),
        in_specs=[
            pl.BlockSpec((bt, 1), lambda i: (i, 0)),
            pl.BlockSpec((bt, hid), lambda i: (i, 0)),
            pl.BlockSpec((bt, hid), lambda i: (i, 0)),
            pl.BlockSpec((n_emb, hid), lambda i: (0, 0)),
            pl.BlockSpec((2 * hid, 4 * hid), lambda i: (0, 0)),
            pl.BlockSpec((1, 4 * hid), lambda i: (0, 0)),
            pl.BlockSpec((None, hid, _LANE_PAD),
                         lambda i: (_HEAD, 0, 0)),
            pl.BlockSpec((None, 1, _LANE_PAD),
                         lambda i: (_HEAD, 0, 0)),
        ],
        out_specs=(
            pl.BlockSpec((bt, _OUT), lambda i: (i, 0)),
            pl.BlockSpec((bt, hid), lambda i: (i, 0)),
            pl.BlockSpec((bt, hid), lambda i: (i, 0)),
        ),
        compiler_params=pltpu.CompilerParams(
            dimension_semantics=("parallel",)),
    )(idx2, h0, c0, embedding, w_lstm, b_lstm, dec_w_pad, dec_b_pad)


def kernel(inputs, h0, c0, embedding, w_lstm, b_lstm, dec_w_pad, dec_b_pad):
    batch = inputs.shape[0]
    hid = h0.shape[1]

    bt = _BT
    while batch % bt:
        bt //= 2

    idx2 = inputs.reshape(batch, 1)
    logits, hx, cx = _run(idx2, h0, c0, embedding, w_lstm, b_lstm,
                          dec_w_pad, dec_b_pad,
                          batch=batch, hid=hid, bt=bt)
    return logits, (hx, cx)


# speedup vs baseline: 1.0768x; 1.0768x over previous
import functools

import jax
import jax.numpy as jnp
from jax.experimental import pallas as pl
from jax.experimental.pallas import tpu as pltpu

_LANE_PAD = 128
_HEAD = 2
_OUT = 4
_INV_TEMP = 1.0 / 5.0
_TANH_C = 2.5
_BT = 384


def _ctrl_kernel(idx_ref, h_ref, c_ref, emb_ref, w_ref, b_ref,
                 decw_ref, decb_ref, logits_ref, hx_ref, cx_ref, *, hid):
    n_emb = emb_ref.shape[0]
    wx = w_ref[:hid, :].astype(jnp.bfloat16)
    wh = w_ref[hid:, :].astype(jnp.bfloat16)

    eg = jnp.dot(emb_ref[...].astype(jnp.bfloat16), wx,
                 preferred_element_type=jnp.float32)
    eg = (eg + b_ref[...]).astype(jnp.bfloat16)
    onehot = (idx_ref[...] == jax.lax.broadcasted_iota(
        jnp.int32, (1, n_emb), 1)).astype(jnp.bfloat16)
    gx = jnp.dot(onehot, eg, preferred_element_type=jnp.float32)

    gh = jnp.dot(h_ref[...].astype(jnp.bfloat16), wh,
                 preferred_element_type=jnp.float32)
    gates = gx + gh

    i_g = jax.nn.sigmoid(gates[:, 0 * hid:1 * hid])
    f_g = jax.nn.sigmoid(gates[:, 1 * hid:2 * hid])
    g_g = jnp.tanh(gates[:, 2 * hid:3 * hid])
    o_g = jax.nn.sigmoid(gates[:, 3 * hid:4 * hid])

    cx = f_g * c_ref[...] + i_g * g_g
    hx = o_g * jnp.tanh(cx)

    logits = (jnp.dot(hx.astype(jnp.bfloat16),
                      decw_ref[...].astype(jnp.bfloat16),
                      preferred_element_type=jnp.float32)
              + decb_ref[...])
    logits = _TANH_C * jnp.tanh(logits * _INV_TEMP)
    logits_ref[...] = logits[:, :_OUT]
    hx_ref[...] = hx
    cx_ref[...] = cx


@functools.partial(jax.jit, static_argnames=("batch", "hid", "bt"))
def _run(idx2, h0, c0, embedding, w_lstm, b_lstm, dec_w_pad, dec_b_pad,
         batch, hid, bt):
    kernel_body = functools.partial(_ctrl_kernel, hid=hid)
    n_emb = embedding.shape[0]
    return pl.pallas_call(
        kernel_body,
        out_shape=(
            jax.ShapeDtypeStruct((batch, _OUT), jnp.float32),
            jax.ShapeDtypeStruct((batch, hid), jnp.float32),
            jax.ShapeDtypeStruct((batch, hid), jnp.float32),
        ),
        grid=(batch // bt,),
        in_specs=[
            pl.BlockSpec((bt, 1), lambda i: (i, 0)),
            pl.BlockSpec((bt, hid), lambda i: (i, 0)),
            pl.BlockSpec((bt, hid), lambda i: (i, 0)),
            pl.BlockSpec((n_emb, hid), lambda i: (0, 0)),
            pl.BlockSpec((2 * hid, 4 * hid), lambda i: (0, 0)),
            pl.BlockSpec((1, 4 * hid), lambda i: (0, 0)),
            pl.BlockSpec((None, hid, _LANE_PAD),
                         lambda i: (_HEAD, 0, 0)),
            pl.BlockSpec((None, 1, _LANE_PAD),
                         lambda i: (_HEAD, 0, 0)),
        ],
        out_specs=(
            pl.BlockSpec((bt, _OUT), lambda i: (i, 0)),
            pl.BlockSpec((bt, hid), lambda i: (i, 0)),
            pl.BlockSpec((bt, hid), lambda i: (i, 0)),
        ),
        compiler_params=pltpu.CompilerParams(
            dimension_semantics=("parallel",)),
    )(idx2, h0, c0, embedding, w_lstm, b_lstm, dec_w_pad, dec_b_pad)


def kernel(inputs, h0, c0, embedding, w_lstm, b_lstm, dec_w_pad, dec_b_pad):
    batch = inputs.shape[0]
    hid = h0.shape[1]

    bt = _BT
    while batch % bt:
        bt //= 2

    idx2 = inputs.reshape(batch, 1)
    logits, hx, cx = _run(idx2, h0, c0, embedding, w_lstm, b_lstm,
                          dec_w_pad, dec_b_pad,
                          batch=batch, hid=hid, bt=bt)
    return logits, (hx, cx)
